# baseline (speedup 1.0000x reference)
import jax
import jax.numpy as jnp
from jax import lax
from jax.experimental import pallas as pl
from jax.experimental.pallas import tpu as pltpu


def _moe_dense_body(x_ref, Wg_ref, W1_ref, b1_ref, W2_ref, b2_ref,
                    out_ref, comb_ref):
    e = pl.program_id(0)
    E = pl.num_programs(0)
    x = x_ref[...]

    @pl.when(e == 0)
    def _init():
        logits = lax.dot_general(x, Wg_ref[...], (((1,), (1,)), ((), ())),
                                 preferred_element_type=jnp.float32)
        s = jax.nn.sigmoid(logits)
        lane = lax.broadcasted_iota(jnp.int32, s.shape, 1)
        m1 = jnp.max(s, axis=1, keepdims=True)
        i1 = jnp.min(jnp.where(s >= m1, lane, E), axis=1, keepdims=True)
        mask1 = lane == i1
        s2 = jnp.where(mask1, -1.0, s)
        m2 = jnp.max(s2, axis=1, keepdims=True)
        i2 = jnp.min(jnp.where(s2 >= m2, lane, E), axis=1, keepdims=True)
        mask2 = lane == i2
        comb_ref[...] = (jnp.where(mask1, m1, 0.0)
                         + jnp.where(mask2, m2, 0.0))
        out_ref[...] = jnp.zeros_like(out_ref)

    onehot = (lax.broadcasted_iota(jnp.int32, (E, 1), 0) == e
              ).astype(jnp.float32)
    w_e = lax.dot_general(comb_ref[...], onehot, (((1,), (0,)), ((), ())),
                          preferred_element_type=jnp.float32)
    h = jnp.maximum(
        lax.dot_general(x, W1_ref[0], (((1,), (0,)), ((), ())),
                        preferred_element_type=jnp.float32)
        + b1_ref[0], 0.0)
    out_ref[...] += (
        lax.dot_general(h * w_e, W2_ref[0], (((1,), (0,)), ((), ())),
                        preferred_element_type=jnp.float32)
        + w_e * b2_ref[0])


def kernel(x, Wg, W1, b1, W2, b2):
    N, D = x.shape
    E, _, H = W1.shape
    b1 = b1.reshape(E, 1, H)
    b2 = b2.reshape(E, 1, D)
    return pl.pallas_call(
        _moe_dense_body,
        grid=(E,),
        in_specs=[
            pl.BlockSpec((N, D), lambda e: (0, 0)),
            pl.BlockSpec((E, D), lambda e: (0, 0)),
            pl.BlockSpec((1, D, H), lambda e: (e, 0, 0)),
            pl.BlockSpec((1, 1, H), lambda e: (e, 0, 0)),
            pl.BlockSpec((1, H, D), lambda e: (e, 0, 0)),
            pl.BlockSpec((1, 1, D), lambda e: (e, 0, 0)),
        ],
        out_specs=pl.BlockSpec((N, D), lambda e: (0, 0)),
        out_shape=jax.ShapeDtypeStruct((N, D), jnp.float32),
        scratch_shapes=[pltpu.VMEM((N, E), jnp.float32)],
        compiler_params=pltpu.CompilerParams(
            dimension_semantics=("arbitrary",)),
    )(x, Wg, W1, b1, W2, b2)


# speedup vs baseline: 1.0596x; 1.0596x over previous
import functools

import jax
import jax.numpy as jnp
from jax import lax
from jax.experimental import pallas as pl
from jax.experimental.pallas import tpu as pltpu

_N, _D, _E, _H, _K = 2048, 768, 64, 128, 2
_B = 64
_NPB = _N * _K // _B + _E - 1
_NPB_PAD = 128
_P = _NPB_PAD * _B


def _shift_down(c, sh):
    return jnp.concatenate(
        [jnp.zeros((sh, c.shape[1]), c.dtype), c[: c.shape[0] - sh]], axis=0)


def _shift_right(c, sh):
    return jnp.concatenate(
        [jnp.zeros((c.shape[0], sh), c.dtype), c[:, : c.shape[1] - sh]],
        axis=1)


def _routing_body(x_ref, Wg_ref, dst_ref, wv_ref, bexp_ref, nact_ref):
    x = x_ref[...]
    N, E, B = _N, _E, _B
    logits = lax.dot_general(x, Wg_ref[...], (((1,), (1,)), ((), ())),
                             preferred_element_type=jnp.float32)
    s = jax.nn.sigmoid(logits)
    lane = lax.broadcasted_iota(jnp.int32, s.shape, 1)
    m1 = jnp.max(s, axis=1, keepdims=True)
    i1 = jnp.min(jnp.where(s >= m1, lane, E), axis=1, keepdims=True)
    mask1 = lane == i1
    s2 = jnp.where(mask1, -1.0, s)
    m2 = jnp.max(s2, axis=1, keepdims=True)
    i2 = jnp.min(jnp.where(s2 >= m2, lane, E), axis=1, keepdims=True)
    mask2 = lane == i2

    hist = mask1.astype(jnp.int32) + mask2.astype(jnp.int32)
    c = hist
    sh = 1
    while sh < N:
        c = c + _shift_down(c, sh)
        sh *= 2
    cum_excl = c - hist
    counts = jnp.sum(hist, axis=0, keepdims=True)
    nb = (counts + B - 1) // B
    cnb = nb
    sh = 1
    while sh < E:
        cnb = cnb + _shift_right(cnb, sh)
        sh *= 2
    nbo = cnb - nb
    nact = jnp.max(cnb, axis=1, keepdims=True)
    poff = B * nbo

    z = jnp.zeros_like(cum_excl)
    rank0 = jnp.sum(jnp.where(mask1, cum_excl, z), axis=1, keepdims=True)
    rank1 = jnp.sum(jnp.where(mask2, cum_excl, z), axis=1, keepdims=True)
    poffb = jnp.broadcast_to(poff, (N, E))
    off0 = jnp.sum(jnp.where(mask1, poffb, z), axis=1, keepdims=True)
    off1 = jnp.sum(jnp.where(mask2, poffb, z), axis=1, keepdims=True)
    dst_ref[...] = jnp.concatenate([off0 + rank0, off1 + rank1], axis=1)
    wv_ref[...] = jnp.concatenate([m1, m2], axis=1)

    bb = lax.broadcasted_iota(jnp.int32, (_NPB_PAD, E), 0)
    bbc = jnp.minimum(bb, nact - 1)
    cmp = jnp.broadcast_to(cnb, (_NPB_PAD, E)) <= bbc
    bexp_ref[...] = jnp.sum(cmp.astype(jnp.int32), axis=1, keepdims=True)
    nact_ref[...] = nact


def _routing(x, Wg):
    return pl.pallas_call(
        _routing_body,
        out_shape=[
            jax.ShapeDtypeStruct((_N, _K), jnp.int32),
            jax.ShapeDtypeStruct((_N, _K), jnp.float32),
            jax.ShapeDtypeStruct((_NPB_PAD, 1), jnp.int32),
            jax.ShapeDtypeStruct((1, 1), jnp.int32),
        ],
    )(x, Wg)


def _ffn_body(bexp_s, nact_s, xs_ref, ws_ref, W1_ref, b1_ref, W2_ref, b2_ref,
              ys_ref):
    b = pl.program_id(0)

    @pl.when(b < nact_s[0])
    def _():
        w = ws_ref[0]
        h = jnp.maximum(
            lax.dot_general(xs_ref[...], W1_ref[0], (((1,), (0,)), ((), ())),
                            preferred_element_type=jnp.float32)
            + b1_ref[0], 0.0)
        ys_ref[...] = (
            lax.dot_general(h * w, W2_ref[0], (((1,), (0,)), ((), ())),
                            preferred_element_type=jnp.float32)
            + w * b2_ref[0])


def _grouped_ffn(bexp, nact, xs, ws, W1, b1, W2, b2):
    D, H = _D, _H
    grid_spec = pltpu.PrefetchScalarGridSpec(
        num_scalar_prefetch=2,
        grid=(_NPB_PAD,),
        in_specs=[
            pl.BlockSpec((_B, D),
                         lambda b, be, na: (jnp.minimum(b, na[0] - 1), 0)),
            pl.BlockSpec((1, _B, 1),
                         lambda b, be, na: (jnp.minimum(b, na[0] - 1), 0, 0)),
            pl.BlockSpec((1, D, H), lambda b, be, na: (be[b], 0, 0)),
            pl.BlockSpec((1, 1, H), lambda b, be, na: (be[b], 0, 0)),
            pl.BlockSpec((1, H, D), lambda b, be, na: (be[b], 0, 0)),
            pl.BlockSpec((1, 1, D), lambda b, be, na: (be[b], 0, 0)),
        ],
        out_specs=pl.BlockSpec(
            (_B, D), lambda b, be, na: (jnp.minimum(b, na[0] - 1), 0)),
    )
    return pl.pallas_call(
        _ffn_body,
        grid_spec=grid_spec,
        out_shape=jax.ShapeDtypeStruct((_P, D), jnp.float32),
        compiler_params=pltpu.CompilerParams(
            dimension_semantics=("arbitrary",)),
    )(bexp, nact, xs, ws.reshape(_NPB_PAD, _B, 1), W1,
      b1.reshape(_E, 1, H), W2, b2.reshape(_E, 1, D))


def kernel(x, Wg, W1, b1, W2, b2):
    dst, wv, bexp, nact = _routing(x, Wg)

    dstf = dst.reshape(-1)
    tid = jnp.arange(_N * _K, dtype=jnp.int32) // _K
    ts = jnp.zeros((_P,), jnp.int32).at[dstf].set(tid)
    ws = jnp.zeros((_P,), jnp.float32).at[dstf].set(wv.reshape(-1))
    xs = x[ts]

    ys = _grouped_ffn(bexp.reshape(-1), nact.reshape(-1), xs, ws,
                      W1, b1, W2, b2)

    out = ys[dst[:, 0]] + ys[dst[:, 1]]
    return out
